# baseline (speedup 1.0000x reference)
import functools

import jax
import jax.numpy as jnp
from jax import lax
from jax.experimental import pallas as pl
from jax.experimental.pallas import tpu as pltpu
from jax.experimental.pallas import tpu_sc as plsc


def _mm(a, b):
    return jax.lax.dot_general(
        a.astype(jnp.bfloat16), b.astype(jnp.bfloat16),
        (((1,), (0,)), ((), ())),
        preferred_element_type=jnp.float32)


def _mmT(a, b):
    return jax.lax.dot_general(
        a.astype(jnp.bfloat16), b.astype(jnp.bfloat16),
        (((0,), (0,)), ((), ())),
        preferred_element_type=jnp.float32)


def _shr(a):
    return jnp.concatenate([jnp.zeros_like(a[:, :1]), a[:, :-1]], axis=1)


def _shl(a):
    return jnp.concatenate([a[:, 1:], jnp.zeros_like(a[:, :1])], axis=1)


def _shd(a):
    return jnp.concatenate([jnp.zeros_like(a[:1, :]), a[:-1, :]], axis=0)


def _shu(a):
    return jnp.concatenate([a[1:, :], jnp.zeros_like(a[:1, :])], axis=0)


def _enc_kernel(xv_ref, w0m_ref, wbig_ref, wez_ref, bias_ref, cb_ref,
                ze_ref, idx_ref, cbn_ref):
    f32 = jnp.float32
    L = 1024
    relu = lambda v: jnp.maximum(v, 0.0)
    W = lambda i, j: wbig_ref[i, j]
    bias = lambda i: bias_ref[256 * i:256 * (i + 1)]
    bz = bias_ref[1792:1920]

    xv = xv_ref[0]
    segs = []
    for p in range(4):
        lo, hi = 2 * p - 1, 2 * p + 3
        if lo < 0:
            seg = jnp.concatenate([_shd(xv[:, 7:8]), xv[:, 0:hi]], axis=1)
        elif hi > 8:
            seg = jnp.concatenate([xv[:, lo:8], _shu(xv[:, 0:1])], axis=1)
        else:
            seg = xv[:, lo:hi]
        segs.append(seg)
    xpm = jnp.concatenate(segs, axis=0)
    h0all = relu(jax.lax.dot_general(
        w0m_ref[...].astype(jnp.bfloat16), xpm.astype(jnp.bfloat16),
        (((1,), (1,)), ((), ())),
        preferred_element_type=f32) + bias(0))
    U = [h0all[:, p * L:(p + 1) * L] for p in range(4)]

    eIdx, oIdx = (1, 2, 3, 0), (3, 0, 1, 2)
    E = jnp.zeros((256, L), f32)
    O = jnp.zeros((256, L), f32)
    for p in range(4):
        ec = _mm(W(0, eIdx[p]), U[p])
        oc = _mm(W(0, oIdx[p]), U[p])
        E = E + (_shr(ec) if p == 3 else ec)
        O = O + (_shl(oc) if p == 0 else oc)
    E = relu(E + bias(1))
    O = relu(O + bias(1))

    h2 = relu(_mm(W(1, 1), E) + _shl(_mm(W(1, 3), E))
              + _shr(_mm(W(1, 0), O)) + _mm(W(1, 2), O) + bias(2))

    ze = (_shr(_mm(wez_ref[0], h2)) + _mm(wez_ref[1], h2)
          + _shl(_mm(wez_ref[2], h2)) + bz)
    ze_ref[0] = ze

    cb = cb_ref[...]
    cbn = cb / (jnp.sqrt(jnp.sum(cb * cb, axis=1, keepdims=True)) + 1e-8)
    cbn_ref[...] = cbn
    zn = ze / (jnp.sqrt(jnp.sum(ze * ze, axis=0, keepdims=True)) + 1e-8)
    S = _mm(cbn, zn)
    m = jnp.max(S, axis=0, keepdims=True)
    kio = jax.lax.broadcasted_iota(jnp.int32, S.shape, 0)
    idx = jnp.min(jnp.where(S >= m, kio, S.shape[0]), axis=0, keepdims=True)
    idx_ref[0] = idx


def _dec_kernel(zr_ref, wbig_ref, wdz_ref, wout_ref, bias_ref,
                y8_ref, zq_ref):
    f32 = jnp.float32
    L = 1024
    relu = lambda v: jnp.maximum(v, 0.0)
    W = lambda i, j: wbig_ref[i, j]
    bias = lambda i: bias_ref[256 * i:256 * (i + 1)]
    bout = bias_ref[1920:1928][:1]

    zq = jnp.transpose(zr_ref[...], (1, 0))
    zq_ref[0] = zq

    g0 = relu(_shr(_mm(wdz_ref[0], zq)) + _mm(wdz_ref[1], zq)
              + _shl(_mm(wdz_ref[2], zq)) + bias(3))

    e1 = relu(_shr(_mmT(W(2, 0), g0)) + _mmT(W(2, 2), g0) + bias(4))
    o1 = relu(_mmT(W(2, 1), g0) + _shl(_mmT(W(2, 3), g0)) + bias(4))

    Re = [_mmT(W(3, j), e1) for j in range(4)]
    Ro = [_mmT(W(3, j), o1) for j in range(4)]
    P = [relu(_shr(Ro[0]) + Re[2] + bias(5)),
         relu(Re[1] + Ro[3] + bias(5)),
         relu(Re[0] + Ro[2] + bias(5)),
         relu(Ro[1] + _shl(Re[3]) + bias(5))]

    Rp = [[_mmT(W(4, j), P[p]) for j in range(4)] for p in range(4)]
    Q = []
    for p in range(4):
        t0 = _shr(Rp[3][0]) if p == 0 else Rp[p - 1][0]
        Q.append(relu(t0 + Rp[p][2] + bias(6)))
        t3 = _shl(Rp[0][3]) if p == 3 else Rp[p + 1][3]
        Q.append(relu(Rp[p][1] + t3 + bias(6)))

    G = jnp.concatenate(Q, axis=1)
    Ro8 = jax.lax.dot_general(
        wout_ref[...].astype(jnp.bfloat16), G.astype(jnp.bfloat16),
        (((0,), (0,)), ((), ())),
        preferred_element_type=f32)
    r0 = [Ro8[0:1, q * L:(q + 1) * L] for q in range(8)]
    r1 = [Ro8[1:2, q * L:(q + 1) * L] for q in range(8)]
    r2 = [Ro8[2:3, q * L:(q + 1) * L] for q in range(8)]
    ys = []
    for q in range(8):
        a = _shr(r0[7]) if q == 0 else r0[q - 1]
        c = _shl(r2[0]) if q == 7 else r2[q + 1]
        ys.append(a + r1[q] + c)
    y8_ref[0] = jnp.concatenate(ys, axis=0) + bout


def _sc_gather(cbn, idx_flat):
    n_rows = idx_flat.shape[0]
    n_w = 32
    b_per_w = n_rows // n_w
    mesh = plsc.VectorSubcoreMesh(core_axis_name="c", subcore_axis_name="s")

    @functools.partial(
        pl.kernel, mesh=mesh,
        out_type=jax.ShapeDtypeStruct((n_rows, 128), jnp.float32),
        scratch_types=[
            pltpu.VMEM((b_per_w,), jnp.int32),
            pltpu.VMEM((b_per_w, 128), jnp.float32),
            pltpu.SemaphoreType.DMA,
        ],
    )
    def k(table_hbm, idx_hbm, out_hbm, idx_v, rows_v, sem):
        wid = lax.axis_index("s") * 2 + lax.axis_index("c")
        base = wid * b_per_w
        pltpu.sync_copy(idx_hbm.at[pl.ds(base, b_per_w)], idx_v)
        pltpu.async_copy(table_hbm.at[idx_v], rows_v, sem).wait()
        pltpu.sync_copy(rows_v, out_hbm.at[pl.ds(base, b_per_w)])

    return k(cbn, idx_flat)


def kernel(x, enc_w0, enc_b0, enc_w1, enc_b1, enc_w2, enc_b2, enc_wz, enc_bz,
           codebook, dec_wz, dec_bz, dec_w0, dec_b0, dec_w1, dec_b1,
           dec_w2, dec_b2, dec_wout, dec_bout):
    B = x.shape[0]
    L = 1024
    f32 = jnp.float32

    xv = x.reshape(B, L, 8)
    w0m = enc_w0.reshape(256, 4)
    wbig = jnp.stack([enc_w1, enc_w2, dec_w0, dec_w1, dec_w2]) \
        .transpose(0, 3, 1, 2)
    wez = enc_wz.transpose(2, 0, 1)
    wdz = dec_wz.transpose(2, 0, 1)
    wout = dec_wout.reshape(256, 3)
    bias = jnp.concatenate(
        [enc_b0, enc_b1, enc_b2, dec_bz, dec_b0, dec_b1, dec_b2,
         enc_bz, dec_bout, jnp.zeros((7,), f32)]).reshape(1928, 1)

    full = lambda a: pl.BlockSpec(a.shape, lambda b: (0,) * a.ndim)
    enc_ins = [xv, w0m, wbig, wez, bias, codebook]
    ze, idx, cbn = pl.pallas_call(
        _enc_kernel,
        grid=(B,),
        in_specs=[pl.BlockSpec((1, L, 8), lambda b: (b, 0, 0))] +
                 [full(a) for a in enc_ins[1:]],
        out_specs=[
            pl.BlockSpec((1, 128, L), lambda b: (b, 0, 0)),
            pl.BlockSpec((1, 1, L), lambda b: (b, 0, 0)),
            pl.BlockSpec((1024, 128), lambda b: (0, 0)),
        ],
        out_shape=[
            jax.ShapeDtypeStruct((B, 128, L), f32),
            jax.ShapeDtypeStruct((B, 1, L), jnp.int32),
            jax.ShapeDtypeStruct((1024, 128), f32),
        ],
    )(*enc_ins)

    zq_rows = _sc_gather(cbn, idx.reshape(B * L))

    dec_ins = [zq_rows, wbig, wdz, wout, bias]
    y8, zq = pl.pallas_call(
        _dec_kernel,
        grid=(B,),
        in_specs=[pl.BlockSpec((L, 128), lambda b: (b, 0))] +
                 [full(a) for a in dec_ins[1:]],
        out_specs=[
            pl.BlockSpec((1, 8, L), lambda b: (b, 0, 0)),
            pl.BlockSpec((1, 128, L), lambda b: (b, 0, 0)),
        ],
        out_shape=[
            jax.ShapeDtypeStruct((B, 8, L), f32),
            jax.ShapeDtypeStruct((B, 128, L), f32),
        ],
    )(*dec_ins)

    x_hat = y8.transpose(0, 2, 1).reshape(B, 1, 8192)
    return (x_hat, ze, zq)


# speedup vs baseline: 1.8483x; 1.8483x over previous
import jax
import jax.numpy as jnp
from jax.experimental import pallas as pl


def _mm(a, b):
    return jax.lax.dot_general(
        a.astype(jnp.bfloat16), b.astype(jnp.bfloat16),
        (((1,), (0,)), ((), ())),
        preferred_element_type=jnp.float32)


def _mmT(a, b):
    return jax.lax.dot_general(
        a.astype(jnp.bfloat16), b.astype(jnp.bfloat16),
        (((0,), (0,)), ((), ())),
        preferred_element_type=jnp.float32)


def _shr(a):
    return jnp.concatenate([jnp.zeros_like(a[:, :1]), a[:, :-1]], axis=1)


def _shl(a):
    return jnp.concatenate([a[:, 1:], jnp.zeros_like(a[:, :1])], axis=1)


def _shd(a):
    return jnp.concatenate([jnp.zeros_like(a[:1, :]), a[:-1, :]], axis=0)


def _shu(a):
    return jnp.concatenate([a[1:, :], jnp.zeros_like(a[:1, :])], axis=0)


def _vqvae_kernel(xv_ref, w0m_ref, wbig_ref, wez_ref, wdz_ref, wout_ref,
                  bias_ref, cb_ref, y8_ref, ze_ref, zq_ref):
    f32 = jnp.float32
    L = 1024
    relu = lambda v: jnp.maximum(v, 0.0)
    W = lambda i, j: wbig_ref[i, j]
    bias = lambda i: bias_ref[256 * i:256 * (i + 1)]
    bz = bias_ref[1792:1920][:128]
    bout = bias_ref[1920:1928][:1]

    xv = xv_ref[0]
    segs = []
    for p in range(4):
        lo, hi = 2 * p - 1, 2 * p + 3
        if lo < 0:
            seg = jnp.concatenate([_shd(xv[:, 7:8]), xv[:, 0:hi]], axis=1)
        elif hi > 8:
            seg = jnp.concatenate([xv[:, lo:8], _shu(xv[:, 0:1])], axis=1)
        else:
            seg = xv[:, lo:hi]
        segs.append(seg)
    xpm = jnp.concatenate(segs, axis=0)
    h0all = relu(jax.lax.dot_general(
        w0m_ref[...].astype(jnp.bfloat16), xpm.astype(jnp.bfloat16),
        (((1,), (1,)), ((), ())),
        preferred_element_type=f32) + bias(0))
    U = [h0all[:, p * L:(p + 1) * L] for p in range(4)]

    eIdx, oIdx = (1, 2, 3, 0), (3, 0, 1, 2)
    E = jnp.zeros((256, L), f32)
    O = jnp.zeros((256, L), f32)
    for p in range(4):
        ec = _mm(W(0, eIdx[p]), U[p])
        oc = _mm(W(0, oIdx[p]), U[p])
        E = E + (_shr(ec) if p == 3 else ec)
        O = O + (_shl(oc) if p == 0 else oc)
    E = relu(E + bias(1))
    O = relu(O + bias(1))

    h2 = relu(_mm(W(1, 1), E) + _shl(_mm(W(1, 3), E))
              + _shr(_mm(W(1, 0), O)) + _mm(W(1, 2), O) + bias(2))

    ze = (_shr(_mm(wez_ref[0], h2)) + _mm(wez_ref[1], h2)
          + _shl(_mm(wez_ref[2], h2)) + bz)
    ze_ref[0] = ze

    cb = cb_ref[...]
    cbn = cb / (jnp.sqrt(jnp.sum(cb * cb, axis=1, keepdims=True)) + 1e-8)
    zn = ze / (jnp.sqrt(jnp.sum(ze * ze, axis=0, keepdims=True)) + 1e-8)
    S = _mm(cbn, zn)
    m = jnp.max(S, axis=0, keepdims=True)
    kio = jax.lax.broadcasted_iota(jnp.int32, S.shape, 0)
    idx = jnp.min(jnp.where(S >= m, kio, S.shape[0]), axis=0, keepdims=True)
    onehot = (kio == idx).astype(jnp.bfloat16)
    c1 = cbn.astype(jnp.bfloat16)
    r1 = cbn - c1.astype(f32)
    c2 = r1.astype(jnp.bfloat16)
    c3 = (r1 - c2.astype(f32)).astype(jnp.bfloat16)
    sel = lambda limb: jax.lax.dot_general(
        limb, onehot, (((0,), (0,)), ((), ())),
        preferred_element_type=f32)
    zq = sel(c1) + sel(c2) + sel(c3)
    zq_ref[0] = zq

    g0 = relu(_shr(_mm(wdz_ref[0], zq)) + _mm(wdz_ref[1], zq)
              + _shl(_mm(wdz_ref[2], zq)) + bias(3))

    e1 = relu(_shr(_mmT(W(2, 0), g0)) + _mmT(W(2, 2), g0) + bias(4))
    o1 = relu(_mmT(W(2, 1), g0) + _shl(_mmT(W(2, 3), g0)) + bias(4))

    Re = [_mmT(W(3, j), e1) for j in range(4)]
    Ro = [_mmT(W(3, j), o1) for j in range(4)]
    P = [relu(_shr(Ro[0]) + Re[2] + bias(5)),
         relu(Re[1] + Ro[3] + bias(5)),
         relu(Re[0] + Ro[2] + bias(5)),
         relu(Ro[1] + _shl(Re[3]) + bias(5))]

    Rp = [[_mmT(W(4, j), P[p]) for j in range(4)] for p in range(4)]
    Q = []
    for p in range(4):
        t0 = _shr(Rp[3][0]) if p == 0 else Rp[p - 1][0]
        Q.append(relu(t0 + Rp[p][2] + bias(6)))
        t3 = _shl(Rp[0][3]) if p == 3 else Rp[p + 1][3]
        Q.append(relu(Rp[p][1] + t3 + bias(6)))

    G = jnp.concatenate(Q, axis=1)
    Ro8 = jax.lax.dot_general(
        wout_ref[...].astype(jnp.bfloat16), G.astype(jnp.bfloat16),
        (((0,), (0,)), ((), ())),
        preferred_element_type=f32)
    r0 = [Ro8[0:1, q * L:(q + 1) * L] for q in range(8)]
    r1 = [Ro8[1:2, q * L:(q + 1) * L] for q in range(8)]
    r2 = [Ro8[2:3, q * L:(q + 1) * L] for q in range(8)]
    ys = []
    for q in range(8):
        a = _shr(r0[7]) if q == 0 else r0[q - 1]
        c = _shl(r2[0]) if q == 7 else r2[q + 1]
        ys.append(a + r1[q] + c)
    y8_ref[0] = jnp.concatenate(ys, axis=0) + bout


def kernel(x, enc_w0, enc_b0, enc_w1, enc_b1, enc_w2, enc_b2, enc_wz, enc_bz,
           codebook, dec_wz, dec_bz, dec_w0, dec_b0, dec_w1, dec_b1,
           dec_w2, dec_b2, dec_wout, dec_bout):
    B = x.shape[0]
    L = 1024
    f32 = jnp.float32

    xv = x.reshape(B, L, 8)
    w0m = enc_w0.reshape(256, 4)
    wbig = jnp.stack([enc_w1, enc_w2, dec_w0, dec_w1, dec_w2]) \
        .transpose(0, 3, 1, 2)
    wez = enc_wz.transpose(2, 0, 1)
    wdz = dec_wz.transpose(2, 0, 1)
    wout = dec_wout.reshape(256, 3)
    bias = jnp.concatenate(
        [enc_b0, enc_b1, enc_b2, dec_bz, dec_b0, dec_b1, dec_b2,
         enc_bz, dec_bout, jnp.zeros((7,), f32)]).reshape(1928, 1)

    full = lambda a: pl.BlockSpec(a.shape, lambda b: (0,) * a.ndim)
    ins = [xv, w0m, wbig, wez, wdz, wout, bias, codebook]
    in_specs = [pl.BlockSpec((1, L, 8), lambda b: (b, 0, 0))] + \
        [full(a) for a in ins[1:]]

    y8, ze, zq = pl.pallas_call(
        _vqvae_kernel,
        grid=(B,),
        in_specs=in_specs,
        out_specs=[
            pl.BlockSpec((1, 8, L), lambda b: (b, 0, 0)),
            pl.BlockSpec((1, 128, L), lambda b: (b, 0, 0)),
            pl.BlockSpec((1, 128, L), lambda b: (b, 0, 0)),
        ],
        out_shape=[
            jax.ShapeDtypeStruct((B, 8, L), f32),
            jax.ShapeDtypeStruct((B, 128, L), f32),
            jax.ShapeDtypeStruct((B, 128, L), f32),
        ],
    )(*ins)

    x_hat = y8.transpose(0, 2, 1).reshape(B, 1, 8192)
    return (x_hat, ze, zq)
